# Initial kernel scaffold; baseline (speedup 1.0000x reference)
import jax
import jax.numpy as jnp
from jax.experimental import pallas as pl


def kernel(x, edge_index, edge_weight, W):
    raise NotImplementedError("write your pallas kernel here")


# speedup vs baseline: 2.4363x; 2.4363x over previous
import functools

import jax
import jax.numpy as jnp
from jax import lax
from jax.experimental import pallas as pl
from jax.experimental.pallas import tpu as pltpu
from jax.experimental.pallas import tpu_sc as plsc

N_NODES = 10000
N_EDGES = 320000
D = 128

NS = 16
EPW = N_EDGES // NS
B = 80
NCHUNK = EPW // B
RPT = 624
TAIL = N_NODES - NS * RPT


def _spmm_sc(x, row, col, w):
    mesh = plsc.VectorSubcoreMesh(core_axis_name="c", subcore_axis_name="s",
                                  num_cores=1)

    @functools.partial(
        pl.kernel,
        mesh=mesh,
        out_type=jax.ShapeDtypeStruct((N_NODES, D), jnp.float32),
        scratch_types=[
            pltpu.VMEM((B,), jnp.int32),
            pltpu.VMEM((B,), jnp.int32),
            pltpu.VMEM((B,), jnp.float32),
            pltpu.VMEM((B, D), jnp.float32),
            pltpu.VMEM_SHARED((N_NODES, D), jnp.float32),
            pltpu.SemaphoreType.DMA,
        ],
    )
    def spmm(x_hbm, row_hbm, col_hbm, w_hbm, out_hbm,
             colv, rowv, wv, rowsv, acc, sem):
        sid = lax.axis_index("s")

        zvec = jnp.zeros((16,), jnp.float32)

        def zero_body(r, carry):
            for j in range(D // 16):
                rowsv[r, pl.ds(j * 16, 16)] = zvec
            return carry

        lax.fori_loop(0, B, zero_body, 0)
        base = sid * RPT
        for k in range(RPT // B):
            pltpu.sync_copy(rowsv, acc.at[pl.ds(base + k * B, B)])
        rem = RPT - (RPT // B) * B
        pltpu.sync_copy(rowsv.at[pl.ds(0, rem)],
                        acc.at[pl.ds(base + RPT - rem, rem)])

        @pl.when(sid == NS - 1)
        def _zero_tail():
            pltpu.sync_copy(rowsv.at[pl.ds(0, TAIL)],
                            acc.at[pl.ds(NS * RPT, TAIL)])

        plsc.subcore_barrier()

        ebase = sid * EPW

        def chunk_body(i, carry):
            off = ebase + i * B
            pltpu.sync_copy(col_hbm.at[pl.ds(off, B)], colv)
            pltpu.sync_copy(row_hbm.at[pl.ds(off, B)], rowv)
            pltpu.sync_copy(w_hbm.at[pl.ds(off, B)], wv)
            pltpu.async_copy(x_hbm.at[colv], rowsv, sem).wait()

            def scale_body(g, c2):
                wchunk = wv[pl.ds(g * 16, 16)]
                for t in range(16):
                    b = g * 16 + t
                    ws = wchunk[t]
                    for j in range(D // 16):
                        sl = pl.ds(j * 16, 16)
                        rowsv[b, sl] = rowsv[b, sl] * ws
                return c2

            lax.fori_loop(0, B // 16, scale_body, 0)
            pltpu.sync_copy(rowsv, acc.at[rowv], add=True)
            return carry

        lax.fori_loop(0, NCHUNK, chunk_body, 0)
        plsc.subcore_barrier()

        sl = pl.ds(sid * RPT, RPT)
        pltpu.sync_copy(acc.at[sl], out_hbm.at[sl])

        @pl.when(sid == NS - 1)
        def _write_tail():
            tl = pl.ds(NS * RPT, TAIL)
            pltpu.sync_copy(acc.at[tl], out_hbm.at[tl])

    return spmm(x, row, col, w)


def _matmul_tc(h, W):
    BM = 2000
    dims = (((1,), (1,)), ((), ()))

    def body(h_ref, w_ref, o_ref):
        o_ref[...] = lax.dot_general(h_ref[...], w_ref[...], dims,
                                     preferred_element_type=jnp.float32)

    return pl.pallas_call(
        body,
        grid=(N_NODES // BM,),
        in_specs=[
            pl.BlockSpec((BM, D), lambda i: (i, 0)),
            pl.BlockSpec((D, D), lambda i: (0, 0)),
        ],
        out_specs=pl.BlockSpec((BM, D), lambda i: (i, 0)),
        out_shape=jax.ShapeDtypeStruct((N_NODES, D), jnp.float32),
    )(h, W)


def kernel(x, edge_index, edge_weight, W):
    row = edge_index[0].astype(jnp.int32)
    col = edge_index[1].astype(jnp.int32)
    h = _spmm_sc(x, row, col, edge_weight)
    return _matmul_tc(h, W)


# speedup vs baseline: 4.2449x; 1.7424x over previous
import functools

import jax
import jax.numpy as jnp
from jax import lax
from jax.experimental import pallas as pl
from jax.experimental.pallas import tpu as pltpu
from jax.experimental.pallas import tpu_sc as plsc

N_NODES = 10000
N_EDGES = 320000
D = 128

NS = 16
EPW = N_EDGES // NS
B = 80
NCHUNK = EPW // B
RPT = 624
TAIL = N_NODES - NS * RPT


def _spmm_sc(x, row, col, w):
    mesh = plsc.VectorSubcoreMesh(core_axis_name="c", subcore_axis_name="s",
                                  num_cores=1)

    @functools.partial(
        pl.kernel,
        mesh=mesh,
        out_type=jax.ShapeDtypeStruct((N_NODES, D), jnp.float32),
        scratch_types=[
            pltpu.VMEM((B,), jnp.int32),
            pltpu.VMEM((B,), jnp.int32),
            pltpu.VMEM((B,), jnp.int32),
            pltpu.VMEM((B,), jnp.int32),
            pltpu.VMEM((B,), jnp.float32),
            pltpu.VMEM((B,), jnp.float32),
            pltpu.VMEM((B, D), jnp.float32),
            pltpu.VMEM((B, D), jnp.float32),
            pltpu.VMEM_SHARED((N_NODES, D), jnp.float32),
            pltpu.SemaphoreType.DMA,
            pltpu.SemaphoreType.DMA,
            pltpu.SemaphoreType.DMA,
            pltpu.SemaphoreType.DMA,
        ],
    )
    def spmm(x_hbm, row_hbm, col_hbm, w_hbm, out_hbm,
             colv0, colv1, rowv0, rowv1, wv0, wv1, rows0, rows1, acc,
             gsem0, gsem1, ssem0, ssem1):
        sid = lax.axis_index("s")
        colv = (colv0, colv1)
        rowv = (rowv0, rowv1)
        wv = (wv0, wv1)
        rowsv = (rows0, rows1)
        gsem = (gsem0, gsem1)
        ssem = (ssem0, ssem1)

        zvec = jnp.zeros((16,), jnp.float32)

        def zero_body(r, carry):
            for j in range(D // 16):
                rows0[r, pl.ds(j * 16, 16)] = zvec
            return carry

        lax.fori_loop(0, B, zero_body, 0)
        base = sid * RPT
        for k in range(RPT // B):
            pltpu.sync_copy(rows0, acc.at[pl.ds(base + k * B, B)])
        rem = RPT - (RPT // B) * B
        pltpu.sync_copy(rows0.at[pl.ds(0, rem)],
                        acc.at[pl.ds(base + RPT - rem, rem)])

        @pl.when(sid == NS - 1)
        def _zero_tail():
            pltpu.sync_copy(rows0.at[pl.ds(0, TAIL)],
                            acc.at[pl.ds(NS * RPT, TAIL)])

        plsc.subcore_barrier()

        ebase = sid * EPW

        def load_idx(ch, b):
            off = ebase + ch * B
            pltpu.sync_copy(col_hbm.at[pl.ds(off, B)], colv[b])
            pltpu.sync_copy(row_hbm.at[pl.ds(off, B)], rowv[b])
            pltpu.sync_copy(w_hbm.at[pl.ds(off, B)], wv[b])

        for b in range(2):
            load_idx(b, b)
            pltpu.async_copy(x_hbm.at[colv[b]], rowsv[b], gsem[b])

        NPAIR = NCHUNK // 2

        def pair_body(g, carry):
            for b in range(2):
                ch = 2 * g + b
                pltpu.make_async_copy(x_hbm.at[colv[b]], rowsv[b],
                                      gsem[b]).wait()

                def scale_body(q, c2, _b=b):
                    wchunk = wv[_b][pl.ds(q * 16, 16)]
                    for t in range(16):
                        r = q * 16 + t
                        ws = wchunk[t]
                        for j in range(D // 16):
                            sl = pl.ds(j * 16, 16)
                            rowsv[_b][r, sl] = rowsv[_b][r, sl] * ws
                    return c2

                lax.fori_loop(0, B // 16, scale_body, 0)
                pltpu.async_copy(rowsv[b], acc.at[rowv[b]], ssem[b],
                                 add=True)

                @pl.when(g < NPAIR - 1)
                def _reload(b=b, ch=ch):
                    off2 = ebase + (ch + 2) * B
                    pltpu.sync_copy(col_hbm.at[pl.ds(off2, B)], colv[b])
                    pltpu.sync_copy(w_hbm.at[pl.ds(off2, B)], wv[b])
                    pltpu.make_async_copy(rowsv[b], acc.at[rowv[b]],
                                          ssem[b]).wait()
                    pltpu.sync_copy(row_hbm.at[pl.ds(off2, B)], rowv[b])
                    pltpu.async_copy(x_hbm.at[colv[b]], rowsv[b], gsem[b])

                @pl.when(g == NPAIR - 1)
                def _drain(b=b):
                    pltpu.make_async_copy(rowsv[b], acc.at[rowv[b]],
                                          ssem[b]).wait()
            return carry

        lax.fori_loop(0, NPAIR, pair_body, 0)
        plsc.subcore_barrier()

        sl = pl.ds(sid * RPT, RPT)
        pltpu.sync_copy(acc.at[sl], out_hbm.at[sl])

        @pl.when(sid == NS - 1)
        def _write_tail():
            tl = pl.ds(NS * RPT, TAIL)
            pltpu.sync_copy(acc.at[tl], out_hbm.at[tl])

    return spmm(x, row, col, w)


def _matmul_tc(h, W):
    BM = 2000
    dims = (((1,), (1,)), ((), ()))

    def body(h_ref, w_ref, o_ref):
        o_ref[...] = lax.dot_general(h_ref[...], w_ref[...], dims,
                                     preferred_element_type=jnp.float32)

    return pl.pallas_call(
        body,
        grid=(N_NODES // BM,),
        in_specs=[
            pl.BlockSpec((BM, D), lambda i: (i, 0)),
            pl.BlockSpec((D, D), lambda i: (0, 0)),
        ],
        out_specs=pl.BlockSpec((BM, D), lambda i: (i, 0)),
        out_shape=jax.ShapeDtypeStruct((N_NODES, D), jnp.float32),
    )(h, W)


def kernel(x, edge_index, edge_weight, W):
    row = edge_index[0].astype(jnp.int32)
    col = edge_index[1].astype(jnp.int32)
    h = _spmm_sc(x, row, col, edge_weight)
    return _matmul_tc(h, W)


# speedup vs baseline: 6.3520x; 1.4964x over previous
import functools

import jax
import jax.numpy as jnp
from jax import lax
from jax.experimental import pallas as pl
from jax.experimental.pallas import tpu as pltpu
from jax.experimental.pallas import tpu_sc as plsc

N_NODES = 10000
N_EDGES = 320000
D = 128

NS = 16
EPW = N_EDGES // NS
B = 40
NCHUNK = EPW // B
NB = 4
NGRP = NCHUNK // NB
LAST = NCHUNK - 1
RPT = 624
TAIL = N_NODES - NS * RPT


def _spmm_sc(x, row, col, w):
    mesh = plsc.VectorSubcoreMesh(core_axis_name="c", subcore_axis_name="s",
                                  num_cores=1)

    @functools.partial(
        pl.kernel,
        mesh=mesh,
        out_type=jax.ShapeDtypeStruct((N_NODES, D), jnp.float32),
        scratch_types=(
            [pltpu.VMEM((B,), jnp.int32) for _ in range(NB)]
            + [pltpu.VMEM((B,), jnp.int32) for _ in range(NB)]
            + [pltpu.VMEM((B + 8,), jnp.float32) for _ in range(NB)]
            + [pltpu.VMEM((B, D), jnp.float32) for _ in range(NB)]
            + [pltpu.VMEM_SHARED((N_NODES, D), jnp.float32)]
            + [pltpu.SemaphoreType.DMA for _ in range(4 * NB)]
            + [pltpu.SemaphoreType.DMA for _ in range(NB)]
        ),
    )
    def spmm(x_hbm, row_hbm, col_hbm, w_hbm, out_hbm, *refs):
        colv = refs[0:NB]
        rowv = refs[NB:2 * NB]
        wv = refs[2 * NB:3 * NB]
        rowsv = refs[3 * NB:4 * NB]
        acc = refs[4 * NB]
        csem = refs[4 * NB + 1:4 * NB + 1 + NB]
        wsem = refs[4 * NB + 1 + NB:4 * NB + 1 + 2 * NB]
        rsem = refs[4 * NB + 1 + 2 * NB:4 * NB + 1 + 3 * NB]
        gsem = refs[4 * NB + 1 + 3 * NB:4 * NB + 1 + 4 * NB]
        ssem = refs[4 * NB + 1 + 4 * NB:4 * NB + 1 + 5 * NB]
        sid = lax.axis_index("s")

        zvec = jnp.zeros((16,), jnp.float32)

        def zero_body(r, carry):
            for j in range(D // 16):
                rowsv[0][r, pl.ds(j * 16, 16)] = zvec
            return carry

        lax.fori_loop(0, B, zero_body, 0)
        base = sid * RPT
        for k in range(RPT // B):
            pltpu.sync_copy(rowsv[0], acc.at[pl.ds(base + k * B, B)])
        rem = RPT - (RPT // B) * B
        pltpu.sync_copy(rowsv[0].at[pl.ds(0, rem)],
                        acc.at[pl.ds(base + RPT - rem, rem)])

        @pl.when(sid == NS - 1)
        def _zero_tail():
            pltpu.sync_copy(rowsv[0].at[pl.ds(0, TAIL)],
                            acc.at[pl.ds(NS * RPT, TAIL)])

        plsc.subcore_barrier()

        ebase = sid * EPW

        def issue_col_w(ch, b):
            off = ebase + ch * B
            pltpu.async_copy(col_hbm.at[pl.ds(off, B)], colv[b], csem[b])
            pltpu.async_copy(w_hbm.at[pl.ds(off, B)],
                             wv[b].at[pl.ds(0, B)], wsem[b])

        def issue_row(ch, b):
            off = ebase + ch * B
            pltpu.async_copy(row_hbm.at[pl.ds(off, B)], rowv[b], rsem[b])

        def wait_col(b):
            pltpu.make_async_copy(col_hbm.at[pl.ds(0, B)], colv[b],
                                  csem[b]).wait()

        def wait_w(b):
            pltpu.make_async_copy(w_hbm.at[pl.ds(0, B)],
                                  wv[b].at[pl.ds(0, B)], wsem[b]).wait()

        def wait_row(b):
            pltpu.make_async_copy(row_hbm.at[pl.ds(0, B)], rowv[b],
                                  rsem[b]).wait()

        def issue_gather(b):
            pltpu.async_copy(x_hbm.at[colv[b]], rowsv[b], gsem[b])

        def wait_gather(b):
            pltpu.make_async_copy(x_hbm.at[colv[b]], rowsv[b],
                                  gsem[b]).wait()

        def issue_scatter(b):
            pltpu.async_copy(rowsv[b], acc.at[rowv[b]], ssem[b], add=True)

        def wait_scatter(b):
            pltpu.make_async_copy(rowsv[b], acc.at[rowv[b]],
                                  ssem[b]).wait()

        for t in range(NB):
            issue_col_w(t, t)
        for t in range(2):
            wait_col(t)
            issue_row(t, t)
            issue_gather(t)

        def scale(b):
            def scale_body(q, c2, _b=b):
                wchunk = wv[_b][pl.ds(q * 16, 16)]
                for t in range(16):
                    r = q * 16 + t
                    ws = wchunk[t]
                    for j in range(D // 16):
                        sl = pl.ds(j * 16, 16)
                        rowsv[_b][r, sl] = rowsv[_b][r, sl] * ws
                return c2

            lax.fori_loop(0, B // 16, scale_body, 0)
            ntail = B - (B // 16) * 16
            wchunk = wv[b][pl.ds(B - ntail, 16)]
            for t in range(ntail):
                r = B - ntail + t
                ws = wchunk[t]
                for j in range(D // 16):
                    sl = pl.ds(j * 16, 16)
                    rowsv[b][r, sl] = rowsv[b][r, sl] * ws

        def group_body(g, carry):
            for t in range(NB):
                c = NB * g + t
                b = t
                bg = (t + 2) % NB
                wait_w(b)
                wait_gather(b)
                scale(b)
                wait_row(b)
                issue_scatter(b)

                @pl.when(c + NB <= LAST)
                def _prefetch(c=c, b=b):
                    issue_col_w(c + NB, b)

                @pl.when(c >= 2)
                def _drain_prev(bg=bg):
                    wait_scatter(bg)

                @pl.when(c + 2 <= LAST)
                def _launch(c=c, bg=bg):
                    wait_col(bg)
                    issue_row(c + 2, bg)
                    issue_gather(bg)
            return carry

        lax.fori_loop(0, NGRP, group_body, 0)
        wait_scatter((LAST - 1) % NB)
        wait_scatter(LAST % NB)
        plsc.subcore_barrier()

        sl = pl.ds(sid * RPT, RPT)
        pltpu.sync_copy(acc.at[sl], out_hbm.at[sl])

        @pl.when(sid == NS - 1)
        def _write_tail():
            tl = pl.ds(NS * RPT, TAIL)
            pltpu.sync_copy(acc.at[tl], out_hbm.at[tl])

    return spmm(x, row, col, w)


def _matmul_tc(h, W):
    BM = 2000
    dims = (((1,), (1,)), ((), ()))

    def body(h_ref, w_ref, o_ref):
        o_ref[...] = lax.dot_general(h_ref[...], w_ref[...], dims,
                                     preferred_element_type=jnp.float32)

    return pl.pallas_call(
        body,
        grid=(N_NODES // BM,),
        in_specs=[
            pl.BlockSpec((BM, D), lambda i: (i, 0)),
            pl.BlockSpec((D, D), lambda i: (0, 0)),
        ],
        out_specs=pl.BlockSpec((BM, D), lambda i: (i, 0)),
        out_shape=jax.ShapeDtypeStruct((N_NODES, D), jnp.float32),
    )(h, W)


def kernel(x, edge_index, edge_weight, W):
    row = edge_index[0].astype(jnp.int32)
    col = edge_index[1].astype(jnp.int32)
    h = _spmm_sc(x, row, col, edge_weight)
    return _matmul_tc(h, W)


# speedup vs baseline: 6.3544x; 1.0004x over previous
import functools

import jax
import jax.numpy as jnp
from jax import lax
from jax.experimental import pallas as pl
from jax.experimental.pallas import tpu as pltpu
from jax.experimental.pallas import tpu_sc as plsc

N_NODES = 10000
N_EDGES = 320000
D = 128

NS = 16
EPW = N_EDGES // NS
B = 40
NCHUNK = EPW // B
NB = 5
NGRP = NCHUNK // NB
LAST = NCHUNK - 1
RPT = 624
TAIL = N_NODES - NS * RPT


def _spmm_sc(x, row, col, w):
    mesh = plsc.VectorSubcoreMesh(core_axis_name="c", subcore_axis_name="s",
                                  num_cores=1)

    @functools.partial(
        pl.kernel,
        mesh=mesh,
        out_type=jax.ShapeDtypeStruct((N_NODES, D), jnp.float32),
        scratch_types=(
            [pltpu.VMEM((B,), jnp.int32) for _ in range(NB)]
            + [pltpu.VMEM((B,), jnp.int32) for _ in range(NB)]
            + [pltpu.VMEM((B + 8,), jnp.float32) for _ in range(NB)]
            + [pltpu.VMEM((B, D), jnp.float32) for _ in range(NB)]
            + [pltpu.VMEM_SHARED((N_NODES, D), jnp.float32)]
            + [pltpu.SemaphoreType.DMA for _ in range(4 * NB)]
            + [pltpu.SemaphoreType.DMA for _ in range(NB)]
        ),
    )
    def spmm(x_hbm, row_hbm, col_hbm, w_hbm, out_hbm, *refs):
        colv = refs[0:NB]
        rowv = refs[NB:2 * NB]
        wv = refs[2 * NB:3 * NB]
        rowsv = refs[3 * NB:4 * NB]
        acc = refs[4 * NB]
        csem = refs[4 * NB + 1:4 * NB + 1 + NB]
        wsem = refs[4 * NB + 1 + NB:4 * NB + 1 + 2 * NB]
        rsem = refs[4 * NB + 1 + 2 * NB:4 * NB + 1 + 3 * NB]
        gsem = refs[4 * NB + 1 + 3 * NB:4 * NB + 1 + 4 * NB]
        ssem = refs[4 * NB + 1 + 4 * NB:4 * NB + 1 + 5 * NB]
        sid = lax.axis_index("s")

        zvec = jnp.zeros((16,), jnp.float32)

        def zero_body(r, carry):
            for j in range(D // 16):
                rowsv[0][r, pl.ds(j * 16, 16)] = zvec
            return carry

        lax.fori_loop(0, B, zero_body, 0)
        base = sid * RPT
        for k in range(RPT // B):
            pltpu.sync_copy(rowsv[0], acc.at[pl.ds(base + k * B, B)])
        rem = RPT - (RPT // B) * B
        pltpu.sync_copy(rowsv[0].at[pl.ds(0, rem)],
                        acc.at[pl.ds(base + RPT - rem, rem)])

        @pl.when(sid == NS - 1)
        def _zero_tail():
            pltpu.sync_copy(rowsv[0].at[pl.ds(0, TAIL)],
                            acc.at[pl.ds(NS * RPT, TAIL)])

        plsc.subcore_barrier()

        ebase = sid * EPW

        def issue_col_w(ch, b):
            off = ebase + ch * B
            pltpu.async_copy(col_hbm.at[pl.ds(off, B)], colv[b], csem[b])
            pltpu.async_copy(w_hbm.at[pl.ds(off, B)],
                             wv[b].at[pl.ds(0, B)], wsem[b])

        def issue_row(ch, b):
            off = ebase + ch * B
            pltpu.async_copy(row_hbm.at[pl.ds(off, B)], rowv[b], rsem[b])

        def wait_col(b):
            pltpu.make_async_copy(col_hbm.at[pl.ds(0, B)], colv[b],
                                  csem[b]).wait()

        def wait_w(b):
            pltpu.make_async_copy(w_hbm.at[pl.ds(0, B)],
                                  wv[b].at[pl.ds(0, B)], wsem[b]).wait()

        def wait_row(b):
            pltpu.make_async_copy(row_hbm.at[pl.ds(0, B)], rowv[b],
                                  rsem[b]).wait()

        def issue_gather(b):
            pltpu.async_copy(x_hbm.at[colv[b]], rowsv[b], gsem[b])

        def wait_gather(b):
            pltpu.make_async_copy(x_hbm.at[colv[b]], rowsv[b],
                                  gsem[b]).wait()

        def issue_scatter(b):
            pltpu.async_copy(rowsv[b], acc.at[rowv[b]], ssem[b], add=True)

        def wait_scatter(b):
            pltpu.make_async_copy(rowsv[b], acc.at[rowv[b]],
                                  ssem[b]).wait()

        for t in range(NB):
            issue_col_w(t, t)
        for t in range(2):
            wait_col(t)
            issue_row(t, t)
            issue_gather(t)

        def scale(b):
            def scale_body(q, c2, _b=b):
                wchunk = wv[_b][pl.ds(q * 16, 16)]
                for t in range(16):
                    r = q * 16 + t
                    ws = wchunk[t]
                    for j in range(D // 16):
                        sl = pl.ds(j * 16, 16)
                        rowsv[_b][r, sl] = rowsv[_b][r, sl] * ws
                return c2

            lax.fori_loop(0, B // 16, scale_body, 0)
            ntail = B - (B // 16) * 16
            wchunk = wv[b][pl.ds(B - ntail, 16)]
            for t in range(ntail):
                r = B - ntail + t
                ws = wchunk[t]
                for j in range(D // 16):
                    sl = pl.ds(j * 16, 16)
                    rowsv[b][r, sl] = rowsv[b][r, sl] * ws

        def group_body(g, carry):
            for t in range(NB):
                c = NB * g + t
                b = t
                bg = (t + 2) % NB
                wait_w(b)
                wait_gather(b)
                scale(b)
                wait_row(b)
                issue_scatter(b)

                @pl.when(c + NB <= LAST)
                def _prefetch(c=c, b=b):
                    issue_col_w(c + NB, b)

                @pl.when(c >= NB - 2)
                def _drain_prev(bg=bg):
                    wait_scatter(bg)

                @pl.when(c + 2 <= LAST)
                def _launch(c=c, bg=bg):
                    wait_col(bg)
                    issue_row(c + 2, bg)
                    issue_gather(bg)
            return carry

        lax.fori_loop(0, NGRP, group_body, 0)
        for k in range(NB - 2):
            wait_scatter((LAST - k) % NB)
        plsc.subcore_barrier()

        sl = pl.ds(sid * RPT, RPT)
        pltpu.sync_copy(acc.at[sl], out_hbm.at[sl])

        @pl.when(sid == NS - 1)
        def _write_tail():
            tl = pl.ds(NS * RPT, TAIL)
            pltpu.sync_copy(acc.at[tl], out_hbm.at[tl])

    return spmm(x, row, col, w)


def _matmul_tc(h, W):
    BM = 2000
    dims = (((1,), (1,)), ((), ()))

    def body(h_ref, w_ref, o_ref):
        o_ref[...] = lax.dot_general(h_ref[...], w_ref[...], dims,
                                     preferred_element_type=jnp.float32)

    return pl.pallas_call(
        body,
        grid=(N_NODES // BM,),
        in_specs=[
            pl.BlockSpec((BM, D), lambda i: (i, 0)),
            pl.BlockSpec((D, D), lambda i: (0, 0)),
        ],
        out_specs=pl.BlockSpec((BM, D), lambda i: (i, 0)),
        out_shape=jax.ShapeDtypeStruct((N_NODES, D), jnp.float32),
    )(h, W)


def kernel(x, edge_index, edge_weight, W):
    row = edge_index[0].astype(jnp.int32)
    col = edge_index[1].astype(jnp.int32)
    h = _spmm_sc(x, row, col, edge_weight)
    return _matmul_tc(h, W)


# speedup vs baseline: 7.2840x; 1.1463x over previous
import functools

import jax
import jax.numpy as jnp
from jax import lax
from jax.experimental import pallas as pl
from jax.experimental.pallas import tpu as pltpu
from jax.experimental.pallas import tpu_sc as plsc

N_NODES = 10000
N_EDGES = 320000
D = 128

NS = 16
EPW = N_EDGES // NS
B = 80
NCHUNK = EPW // B
NCT = N_EDGES // B
NB = 3
NE = 6
NGRP = (NCHUNK - 4) // NE
LAST = NCHUNK - 1
RPT = 624
TAIL = N_NODES - NS * RPT


def _spmm_sc(x, e_packed, w):
    mesh = plsc.VectorSubcoreMesh(core_axis_name="c", subcore_axis_name="s",
                                  num_cores=1)

    @functools.partial(
        pl.kernel,
        mesh=mesh,
        out_type=jax.ShapeDtypeStruct((N_NODES, D), jnp.float32),
        scratch_types=(
            [pltpu.VMEM((2, B), jnp.int32) for _ in range(NE)]
            + [pltpu.VMEM((B,), jnp.float32) for _ in range(NE)]
            + [pltpu.VMEM((B, D), jnp.float32) for _ in range(NB)]
            + [pltpu.VMEM_SHARED((N_NODES, D), jnp.float32)]
            + [pltpu.SemaphoreType.DMA for _ in range(NE)]
            + [pltpu.SemaphoreType.DMA for _ in range(NB)]
            + [pltpu.SemaphoreType.DMA for _ in range(NB)]
        ),
    )
    def spmm(x_hbm, e_hbm, w_hbm, out_hbm, *refs):
        ebuf = refs[0:NE]
        wbuf = refs[NE:2 * NE]
        rowsv = refs[2 * NE:2 * NE + NB]
        acc = refs[2 * NE + NB]
        sems = refs[2 * NE + NB + 1:]
        esem = sems[0:NE]
        gsem = sems[NE:NE + NB]
        ssem = sems[NE + NB:NE + 2 * NB]
        sid = lax.axis_index("s")

        zvec = jnp.zeros((16,), jnp.float32)

        def zero_body(r, carry):
            for j in range(D // 16):
                rowsv[0][r, pl.ds(j * 16, 16)] = zvec
            return carry

        lax.fori_loop(0, B, zero_body, 0)
        base = sid * RPT
        for k in range(RPT // B):
            pltpu.sync_copy(rowsv[0], acc.at[pl.ds(base + k * B, B)])
        rem = RPT - (RPT // B) * B
        pltpu.sync_copy(rowsv[0].at[pl.ds(0, rem)],
                        acc.at[pl.ds(base + RPT - rem, rem)])

        @pl.when(sid == NS - 1)
        def _zero_tail():
            pltpu.sync_copy(rowsv[0].at[pl.ds(0, TAIL)],
                            acc.at[pl.ds(NS * RPT, TAIL)])

        plsc.subcore_barrier()

        cbase = sid * NCHUNK

        def issue_e(ch, j):
            pltpu.async_copy(e_hbm.at[cbase + ch], ebuf[j], esem[j])
            off = (cbase + ch) * B
            pltpu.async_copy(w_hbm.at[pl.ds(off, B)], wbuf[j], esem[j])

        def wait_e(j):
            pltpu.make_async_copy(e_hbm.at[0], ebuf[j], esem[j]).wait()
            pltpu.make_async_copy(w_hbm.at[pl.ds(0, B)], wbuf[j],
                                  esem[j]).wait()

        def issue_gather(j, b):
            pltpu.async_copy(x_hbm.at[ebuf[j].at[0]], rowsv[b], gsem[b])

        def wait_gather(j, b):
            pltpu.make_async_copy(x_hbm.at[ebuf[j].at[0]], rowsv[b],
                                  gsem[b]).wait()

        def issue_scatter(j, b):
            pltpu.async_copy(rowsv[b], acc.at[ebuf[j].at[1]], ssem[b],
                             add=True)

        def wait_scatter(j, b):
            pltpu.make_async_copy(rowsv[b], acc.at[ebuf[j].at[1]],
                                  ssem[b]).wait()

        def scale(j, b):
            def scale_body(q, c2, _j=j, _b=b):
                wchunk = wbuf[_j][pl.ds(q * 16, 16)]
                for t in range(16):
                    r = q * 16 + t
                    ws = wchunk[t]
                    for f in range(D // 16):
                        sl = pl.ds(f * 16, 16)
                        rowsv[_b][r, sl] = rowsv[_b][r, sl] * ws
                return c2

            lax.fori_loop(0, B // 16, scale_body, 0)

        def step(c, be, b, in_main):
            wait_gather(be, b)
            scale(be, b)
            issue_scatter(be, b)
            if in_main:
                @pl.when(c >= 1)
                def _drain():
                    wait_scatter((be + 5) % NE, (b + 2) % NB)

                issue_e(c + 4, (be + 4) % NE)
                wait_e((be + 2) % NE)
                issue_gather((be + 2) % NE, (b + 2) % NB)
            return c

        for t in range(4):
            issue_e(t, t)
        for t in range(2):
            wait_e(t)
            issue_gather(t, t)

        def group_body(g, carry):
            for t in range(NE):
                step(NE * g + t, t, t % NB, True)
            return carry

        lax.fori_loop(0, NGRP, group_body, 0)

        for (c, be, b) in ((246, 0, 0), (247, 1, 1), (248, 2, 2),
                           (249, 3, 0)):
            wait_gather(be, b)
            scale(be, b)
            issue_scatter(be, b)
            wait_scatter((be + 5) % NE, (b + 2) % NB)
            if c < 248:
                wait_e(be + 2)
                issue_gather(be + 2, (b + 2) % NB)
        wait_scatter(3, 0)
        plsc.subcore_barrier()

        sl = pl.ds(sid * RPT, RPT)
        pltpu.sync_copy(acc.at[sl], out_hbm.at[sl])

        @pl.when(sid == NS - 1)
        def _write_tail():
            tl = pl.ds(NS * RPT, TAIL)
            pltpu.sync_copy(acc.at[tl], out_hbm.at[tl])

    return spmm(x, e_packed, w)


def _matmul_tc(h, W):
    BM = 2000
    dims = (((1,), (1,)), ((), ()))

    def body(h_ref, w_ref, o_ref):
        o_ref[...] = lax.dot_general(h_ref[...], w_ref[...], dims,
                                     preferred_element_type=jnp.float32)

    return pl.pallas_call(
        body,
        grid=(N_NODES // BM,),
        in_specs=[
            pl.BlockSpec((BM, D), lambda i: (i, 0)),
            pl.BlockSpec((D, D), lambda i: (0, 0)),
        ],
        out_specs=pl.BlockSpec((BM, D), lambda i: (i, 0)),
        out_shape=jax.ShapeDtypeStruct((N_NODES, D), jnp.float32),
    )(h, W)


def kernel(x, edge_index, edge_weight, W):
    row = edge_index[0].astype(jnp.int32)
    col = edge_index[1].astype(jnp.int32)
    e_packed = jnp.stack(
        [col.reshape(NCT, B), row.reshape(NCT, B)], axis=1)
    h = _spmm_sc(x, e_packed, edge_weight)
    return _matmul_tc(h, W)
